# baseline (speedup 1.0000x reference)
import functools

import jax
import jax.numpy as jnp
from jax import lax
from jax.experimental import pallas as pl
from jax.experimental.pallas import tpu as pltpu
from jax.experimental.pallas import tpu_sc as plsc

_B = 256
_T = 50
_H = 64
_EMB = 32
_K = 20
_KP1 = 21
_NW = 32
_CBLK = 1024
_INF = float("inf")
_BIGI = 0x7FFFFFFF


def _sc_row_gather(t1, i1, t2, i2):
    tot = i1.shape[0]
    n = tot // _NW
    mesh = plsc.VectorSubcoreMesh(core_axis_name="c", subcore_axis_name="s")

    @functools.partial(
        pl.kernel, mesh=mesh,
        out_type=[jax.ShapeDtypeStruct((tot, 128), jnp.float32),
                  jax.ShapeDtypeStruct((tot, 128), jnp.float32)],
        scratch_types=[pltpu.VMEM((n,), jnp.int32),
                       pltpu.VMEM((n, 128), jnp.float32),
                       pltpu.SemaphoreType.DMA],
    )
    def k(t1h, i1h, t2h, i2h, o1, o2, idx_v, rows_v, sem):
        wid = lax.axis_index("s") * 2 + lax.axis_index("c")
        base = wid * n
        for th, ih, oh in ((t1h, i1h, o1), (t2h, i2h, o2)):
            pltpu.sync_copy(ih.at[pl.ds(base, n)], idx_v)
            pltpu.async_copy(th.at[idx_v], rows_v, sem).wait()
            pltpu.sync_copy(rows_v, oh.at[pl.ds(base, n)])

    return k(t1, i1, t2, i2)


def _topk_body(qrows_ref, qmod_ref, e_ref, y_ref, bv_ref, ys_ref,
               v_ref, *, size, nblk):
    j = pl.program_id(0)

    @pl.when(j == 0)
    def _init():
        bv_ref[...] = jnp.full((_B, _KP1), _INF, jnp.float32)
        ys_ref[...] = jnp.zeros((_B, _KP1), jnp.float32)

    qmod = qmod_ref[...]
    q = jnp.zeros((_B, _EMB), jnp.float32)
    for kq in range(4):
        q = q + jnp.where(qmod == kq,
                          qrows_ref[:, kq * _EMB:(kq + 1) * _EMB], 0.0)
    e = e_ref[...]
    qq = jnp.sum(q * q, axis=1, keepdims=True)
    qr = q.astype(jnp.bfloat16).astype(jnp.float32)
    er = e.astype(jnp.bfloat16).astype(jnp.float32)
    sc = lax.dot_general(qr, er, (((1,), (1,)), ((), ())),
                         preferred_element_type=jnp.float32)
    e2 = e * e
    h1 = e2.astype(jnp.bfloat16).astype(jnp.float32)
    r1 = e2 - h1
    h2 = r1.astype(jnp.bfloat16).astype(jnp.float32)
    h3 = (r1 - h2).astype(jnp.bfloat16).astype(jnp.float32)
    ea = jnp.concatenate([h1, h2, h3], axis=1)
    ee = lax.dot_general(jnp.ones((1, 3 * _EMB), jnp.float32), ea,
                         (((1,), (1,)), ((), ())),
                         preferred_element_type=jnp.float32)
    col = jax.lax.broadcasted_iota(jnp.int32, (_B, _CBLK), 1) + j * _CBLK
    d2 = (qq + ee) - 2.0 * sc
    d2 = jnp.where(col < size, d2, _INF)
    v_ref[...] = d2

    t = jnp.max(bv_ref[...], axis=1, keepdims=True)
    cnt = jnp.max(jnp.sum((d2 < t).astype(jnp.int32), axis=1, keepdims=True))
    nit = jnp.minimum(cnt, _KP1)

    lane = jax.lax.broadcasted_iota(jnp.int32, (_B, _KP1), 1)

    def body(_, carry):
        v = v_ref[...]
        m = jnp.min(v, axis=1, keepdims=True)
        cidx = jnp.min(jnp.where(v == m, col, _BIGI), axis=1, keepdims=True)
        hit = col == cidx
        v_ref[...] = jnp.where(hit, _INF, v)
        yval = jnp.sum(jnp.where(hit, y_ref[...], 0.0), axis=1, keepdims=True)
        bv = bv_ref[...]
        tc = jnp.max(bv, axis=1, keepdims=True)
        slot = jnp.min(jnp.where(bv == tc, lane, _BIGI), axis=1, keepdims=True)
        put = (lane == slot) & (m < tc)
        bv_ref[...] = jnp.where(put, m, bv)
        ys_ref[...] = jnp.where(put, yval, ys_ref[...])
        return carry

    lax.fori_loop(0, nit, body, 0)


def _topk(qrows, qmod, emb, y_ctx):
    size = emb.shape[0]
    nblk = pl.cdiv(size, _CBLK)
    return pl.pallas_call(
        functools.partial(_topk_body, size=size, nblk=nblk),
        grid=(nblk,),
        in_specs=[pl.BlockSpec((_B, 128), lambda j: (0, 0)),
                  pl.BlockSpec((_B, 1), lambda j: (0, 0)),
                  pl.BlockSpec((_CBLK, _EMB), lambda j: (j, 0)),
                  pl.BlockSpec((_B, _CBLK), lambda j: (0, j))],
        out_specs=[pl.BlockSpec((_B, _KP1), lambda j: (0, 0)),
                   pl.BlockSpec((_B, _KP1), lambda j: (0, 0))],
        out_shape=[jax.ShapeDtypeStruct((_B, _KP1), jnp.float32),
                   jax.ShapeDtypeStruct((_B, _KP1), jnp.float32)],
        scratch_shapes=[pltpu.VMEM((_B, _CBLK), jnp.float32)],
        compiler_params=pltpu.CompilerParams(
            dimension_semantics=("arbitrary",)),
    )(qrows, qmod, emb, y_ctx)


def _gru_body(xl_ref, xr_ref,
              wi0l, wh0l, bi0l, bh0l, wi1l, wh1l, bi1l, bh1l,
              wi0r, wh0r, bi0r, bh0r, wi1r, wh1r, bi1r, bh1r,
              wmean, bmean, wstd, bstd, ytr_ref,
              mean_o, std_o, err1_o):

    def gru_side(x_ref, wi0, wh0, bi0, bh0, wi1, wh1, bi1, bh1):
        wi0v, wh0v, bi0v, bh0v = wi0[...], wh0[...], bi0[...], bh0[...]
        wi1v, wh1v, bi1v, bh1v = wi1[...], wh1[...], bi1[...], bh1[...]

        def cell(gx, gh, h):
            r = jax.nn.sigmoid(gx[0:_H] + gh[0:_H])
            z = jax.nn.sigmoid(gx[_H:2 * _H] + gh[_H:2 * _H])
            n = jnp.tanh(gx[2 * _H:] + r * gh[2 * _H:])
            return (1.0 - z) * n + z * h

        def step(tstep, carry):
            h0, h1 = carry
            xt = x_ref[pl.ds(tstep, 1), :]
            gx0 = wi0v * xt + bi0v
            gh0 = lax.dot_general(wh0v, h0, (((1,), (0,)), ((), ())),
                                  preferred_element_type=jnp.float32) + bh0v
            h0n = cell(gx0, gh0, h0)
            gx1 = lax.dot_general(wi1v, h0n, (((1,), (0,)), ((), ())),
                                  preferred_element_type=jnp.float32) + bi1v
            gh1 = lax.dot_general(wh1v, h1, (((1,), (0,)), ((), ())),
                                  preferred_element_type=jnp.float32) + bh1v
            h1n = cell(gx1, gh1, h1)
            return (h0n, h1n)

        z0 = jnp.zeros((_H, _B), jnp.float32)
        _, h1 = lax.fori_loop(0, _T, step, (z0, z0))
        return h1

    hl = gru_side(xl_ref, wi0l, wh0l, bi0l, bh0l, wi1l, wh1l, bi1l, bh1l)
    hr = gru_side(xr_ref, wi0r, wh0r, bi0r, bh0r, wi1r, wh1r, bi1r, bh1r)
    temp = jnp.concatenate([hl, hr], axis=0)
    mean_ts = jnp.sum(wmean[...] * temp, axis=0, keepdims=True) + bmean[0, 0]
    std_ts = jnp.sum(wstd[...] * temp, axis=0, keepdims=True) + bstd[0, 0]
    mean_o[...] = mean_ts
    std_o[...] = std_ts
    yv = ytr_ref[...]
    e1 = (yv - mean_ts) ** 2 / jnp.exp(std_ts) + std_ts
    err1_o[...] = jnp.sum(e1, axis=1, keepdims=True) / _B


def _gru_call(xl, xr, wp, ytr):
    args = [xl, xr] + wp + [ytr]
    return pl.pallas_call(
        _gru_body,
        out_shape=[jax.ShapeDtypeStruct((1, _B), jnp.float32),
                   jax.ShapeDtypeStruct((1, _B), jnp.float32),
                   jax.ShapeDtypeStruct((1, 1), jnp.float32)],
    )(*args)


def _head_body(bv1_ref, s1_ref, bv2_ref, s2_ref, mts_ref, sts_ref, yc_ref,
               w1_ref, b1_ref, wmo_ref, bmo_ref, wso_ref, bso_ref,
               err2_o, meano_o):

    lane = jax.lax.broadcasted_iota(jnp.int32, (_B, _KP1), 1)

    def side_feats(bv, ysel):
        m = jnp.min(bv, axis=1, keepdims=True)
        sslot = jnp.min(jnp.where(bv == m, lane, _BIGI), axis=1, keepdims=True)
        keep = lane != sslot
        w = jnp.exp(-jnp.sqrt(jnp.maximum(bv, 0.0)))
        wk = jnp.where(keep, w, 0.0)
        sk = jnp.where(keep, ysel, 0.0)
        ws = jnp.sum(wk, axis=1, keepdims=True)
        wm = jnp.sum(sk * wk, axis=1, keepdims=True) / ws
        mean_s = jnp.sum(sk, axis=1, keepdims=True) / _K
        var = jnp.sum(jnp.where(keep, (ysel - mean_s) ** 2, 0.0),
                      axis=1, keepdims=True) / (_K - 1)
        return wm, ws, jnp.sqrt(var)

    wm1, ws1, st1 = side_feats(bv1_ref[...], s1_ref[...])
    wm2, ws2, st2 = side_feats(bv2_ref[...], s2_ref[...])
    mts = mts_ref[...]
    sts = sts_ref[...]
    feats = jnp.concatenate([wm1, ws1, st1, wm2, ws2, st2, mts, sts], axis=1)
    h = lax.dot_general(feats, w1_ref[...], (((1,), (1,)), ((), ())),
                        preferred_element_type=jnp.float32) + b1_ref[...]
    h = jnp.maximum(h, 0.0)
    mean_o = jnp.sum(h * wmo_ref[...], axis=1, keepdims=True) + bmo_ref[0, 0]
    std_o = jnp.sum(h * wso_ref[...], axis=1, keepdims=True) + bso_ref[0, 0]
    yc = yc_ref[...]
    e2 = (yc - mean_o) ** 2 / jnp.exp(std_o) + std_o
    err2_o[...] = jnp.sum(e2, axis=0, keepdims=True) / _B
    meano_o[...] = mean_o


def _head_call(bv1, s1, bv2, s2, mts, sts, ycol, w1, b1, wmo, bmo, wso, bso):
    args = [bv1, s1, bv2, s2, mts, sts, ycol, w1, b1, wmo, bmo, wso, bso]
    return pl.pallas_call(
        _head_body,
        out_shape=[jax.ShapeDtypeStruct((1, 1), jnp.float32),
                   jax.ShapeDtypeStruct((_B, 1), jnp.float32)],
    )(*args)


def kernel(params, x_left, x_right, y, index1, index2, y1_context, y2_context):
    p = params
    size1 = p['emb1'].shape[0]
    size2 = p['emb2'].shape[0]

    i1 = index1.astype(jnp.int32)
    i2 = index2.astype(jnp.int32)
    qrows1, qrows2 = _sc_row_gather(p['emb1'].reshape(-1, 128), i1 >> 2,
                                    p['emb2'].reshape(-1, 128), i2 >> 2)
    qmod1 = (i1 & 3).reshape(_B, 1)
    qmod2 = (i2 & 3).reshape(_B, 1)

    bv1, s1 = _topk(qrows1, qmod1, p['emb1'], y1_context)
    bv2, s2 = _topk(qrows2, qmod2, p['emb2'], y2_context)

    def side_w(side):
        return [p['W_ih_%s0' % side], p['W_hh_%s0' % side],
                p['b_ih_%s0' % side].reshape(3 * _H, 1),
                p['b_hh_%s0' % side].reshape(3 * _H, 1),
                p['W_ih_%s1' % side], p['W_hh_%s1' % side],
                p['b_ih_%s1' % side].reshape(3 * _H, 1),
                p['b_hh_%s1' % side].reshape(3 * _H, 1)]

    wp = (side_w('left') + side_w('right') +
          [p['W_mean'].reshape(2 * _H, 1), p['b_mean'].reshape(1, 1),
           p['W_std'].reshape(2 * _H, 1), p['b_std'].reshape(1, 1)])
    ytr = y.reshape(1, _B)
    mts_tr, sts_tr, err1 = _gru_call(x_left.T, x_right.T, wp, ytr)

    err2, mean_o = _head_call(
        bv1, s1, bv2, s2,
        mts_tr.reshape(_B, 1), sts_tr.reshape(_B, 1), y.reshape(_B, 1),
        p['W_out1'], p['b_out1'].reshape(1, _H),
        p['W_mo'], p['b_mo'].reshape(1, 1),
        p['W_so'], p['b_so'].reshape(1, 1))

    return (err1.reshape(()), err2.reshape(()), mean_o)


# speedup vs baseline: 1.0284x; 1.0284x over previous
import functools

import jax
import jax.numpy as jnp
from jax import lax
from jax.experimental import pallas as pl
from jax.experimental.pallas import tpu as pltpu
from jax.experimental.pallas import tpu_sc as plsc

_B = 256
_T = 50
_H = 64
_EMB = 32
_K = 20
_KP1 = 21
_NW = 32
_CBLK = 2048
_INF = float("inf")
_BIGI = 0x7FFFFFFF


def _sc_row_gather(t1, i1, t2, i2):
    tot = i1.shape[0]
    n = tot // _NW
    mesh = plsc.VectorSubcoreMesh(core_axis_name="c", subcore_axis_name="s")

    @functools.partial(
        pl.kernel, mesh=mesh,
        out_type=[jax.ShapeDtypeStruct((tot, 128), jnp.float32),
                  jax.ShapeDtypeStruct((tot, 128), jnp.float32)],
        scratch_types=[pltpu.VMEM((n,), jnp.int32),
                       pltpu.VMEM((n, 128), jnp.float32),
                       pltpu.SemaphoreType.DMA],
    )
    def k(t1h, i1h, t2h, i2h, o1, o2, idx_v, rows_v, sem):
        wid = lax.axis_index("s") * 2 + lax.axis_index("c")
        base = wid * n
        for th, ih, oh in ((t1h, i1h, o1), (t2h, i2h, o2)):
            pltpu.sync_copy(ih.at[pl.ds(base, n)], idx_v)
            pltpu.async_copy(th.at[idx_v], rows_v, sem).wait()
            pltpu.sync_copy(rows_v, oh.at[pl.ds(base, n)])

    return k(t1, i1, t2, i2)


def _topk_body(qrows_ref, qmod_ref, e_ref, y_ref, bv_ref, ys_ref,
               v_ref, *, size, nblk):
    j = pl.program_id(0)

    @pl.when(j == 0)
    def _init():
        bv_ref[...] = jnp.full((_B, _KP1), _INF, jnp.float32)
        ys_ref[...] = jnp.zeros((_B, _KP1), jnp.float32)

    qmod = qmod_ref[...]
    q = jnp.zeros((_B, _EMB), jnp.float32)
    for kq in range(4):
        q = q + jnp.where(qmod == kq,
                          qrows_ref[:, kq * _EMB:(kq + 1) * _EMB], 0.0)
    e = e_ref[...]
    qq = jnp.sum(q * q, axis=1, keepdims=True)
    qr = q.astype(jnp.bfloat16).astype(jnp.float32)
    er = e.astype(jnp.bfloat16).astype(jnp.float32)
    sc = lax.dot_general(qr, er, (((1,), (1,)), ((), ())),
                         preferred_element_type=jnp.float32)
    e2 = e * e
    h1 = e2.astype(jnp.bfloat16).astype(jnp.float32)
    r1 = e2 - h1
    h2 = r1.astype(jnp.bfloat16).astype(jnp.float32)
    h3 = (r1 - h2).astype(jnp.bfloat16).astype(jnp.float32)
    ea = jnp.concatenate([h1, h2, h3], axis=1)
    ee = lax.dot_general(jnp.ones((1, 3 * _EMB), jnp.float32), ea,
                         (((1,), (1,)), ((), ())),
                         preferred_element_type=jnp.float32)
    col = jax.lax.broadcasted_iota(jnp.int32, (_B, _CBLK), 1) + j * _CBLK
    d2 = (qq + ee) - 2.0 * sc
    d2 = jnp.where(col < size, d2, _INF)
    v_ref[...] = d2

    t = jnp.max(bv_ref[...], axis=1, keepdims=True)
    cnt = jnp.max(jnp.sum((d2 < t).astype(jnp.int32), axis=1, keepdims=True))
    nit = jnp.minimum(cnt, _KP1)

    lane = jax.lax.broadcasted_iota(jnp.int32, (_B, _KP1), 1)

    def body(_, carry):
        v = v_ref[...]
        m = jnp.min(v, axis=1, keepdims=True)
        cidx = jnp.min(jnp.where(v == m, col, _BIGI), axis=1, keepdims=True)
        hit = col == cidx
        v_ref[...] = jnp.where(hit, _INF, v)
        yval = jnp.sum(jnp.where(hit, y_ref[...], 0.0), axis=1, keepdims=True)
        bv = bv_ref[...]
        tc = jnp.max(bv, axis=1, keepdims=True)
        slot = jnp.min(jnp.where(bv == tc, lane, _BIGI), axis=1, keepdims=True)
        put = (lane == slot) & (m < tc)
        bv_ref[...] = jnp.where(put, m, bv)
        ys_ref[...] = jnp.where(put, yval, ys_ref[...])
        return carry

    lax.fori_loop(0, nit, body, 0)


def _topk(qrows, qmod, emb, y_ctx):
    size = emb.shape[0]
    nblk = pl.cdiv(size, _CBLK)
    return pl.pallas_call(
        functools.partial(_topk_body, size=size, nblk=nblk),
        grid=(nblk,),
        in_specs=[pl.BlockSpec((_B, 128), lambda j: (0, 0)),
                  pl.BlockSpec((_B, 1), lambda j: (0, 0)),
                  pl.BlockSpec((_CBLK, _EMB), lambda j: (j, 0)),
                  pl.BlockSpec((_B, _CBLK), lambda j: (0, j))],
        out_specs=[pl.BlockSpec((_B, _KP1), lambda j: (0, 0)),
                   pl.BlockSpec((_B, _KP1), lambda j: (0, 0))],
        out_shape=[jax.ShapeDtypeStruct((_B, _KP1), jnp.float32),
                   jax.ShapeDtypeStruct((_B, _KP1), jnp.float32)],
        scratch_shapes=[pltpu.VMEM((_B, _CBLK), jnp.float32)],
        compiler_params=pltpu.CompilerParams(
            dimension_semantics=("arbitrary",)),
    )(qrows, qmod, emb, y_ctx)


def _gru_body(xl_ref, xr_ref,
              wi0l, wh0l, bi0l, bh0l, wi1l, wh1l, bi1l, bh1l,
              wi0r, wh0r, bi0r, bh0r, wi1r, wh1r, bi1r, bh1r,
              wmean, bmean, wstd, bstd, ytr_ref,
              mean_o, std_o, err1_o):

    def gru_side(x_ref, wi0, wh0, bi0, bh0, wi1, wh1, bi1, bh1):
        wi0v, wh0v, bi0v, bh0v = wi0[...], wh0[...], bi0[...], bh0[...]
        wi1v, wh1v, bi1v, bh1v = wi1[...], wh1[...], bi1[...], bh1[...]

        def cell(gx, gh, h):
            r = jax.nn.sigmoid(gx[0:_H] + gh[0:_H])
            z = jax.nn.sigmoid(gx[_H:2 * _H] + gh[_H:2 * _H])
            n = jnp.tanh(gx[2 * _H:] + r * gh[2 * _H:])
            return (1.0 - z) * n + z * h

        def step(tstep, carry):
            h0, h1 = carry
            xt = x_ref[pl.ds(tstep, 1), :]
            gx0 = wi0v * xt + bi0v
            gh0 = lax.dot_general(wh0v, h0, (((1,), (0,)), ((), ())),
                                  preferred_element_type=jnp.float32) + bh0v
            h0n = cell(gx0, gh0, h0)
            gx1 = lax.dot_general(wi1v, h0n, (((1,), (0,)), ((), ())),
                                  preferred_element_type=jnp.float32) + bi1v
            gh1 = lax.dot_general(wh1v, h1, (((1,), (0,)), ((), ())),
                                  preferred_element_type=jnp.float32) + bh1v
            h1n = cell(gx1, gh1, h1)
            return (h0n, h1n)

        z0 = jnp.zeros((_H, _B), jnp.float32)
        _, h1 = lax.fori_loop(0, _T, step, (z0, z0))
        return h1

    hl = gru_side(xl_ref, wi0l, wh0l, bi0l, bh0l, wi1l, wh1l, bi1l, bh1l)
    hr = gru_side(xr_ref, wi0r, wh0r, bi0r, bh0r, wi1r, wh1r, bi1r, bh1r)
    temp = jnp.concatenate([hl, hr], axis=0)
    mean_ts = jnp.sum(wmean[...] * temp, axis=0, keepdims=True) + bmean[0, 0]
    std_ts = jnp.sum(wstd[...] * temp, axis=0, keepdims=True) + bstd[0, 0]
    mean_o[...] = mean_ts
    std_o[...] = std_ts
    yv = ytr_ref[...]
    e1 = (yv - mean_ts) ** 2 / jnp.exp(std_ts) + std_ts
    err1_o[...] = jnp.sum(e1, axis=1, keepdims=True) / _B


def _gru_call(xl, xr, wp, ytr):
    args = [xl, xr] + wp + [ytr]
    return pl.pallas_call(
        _gru_body,
        out_shape=[jax.ShapeDtypeStruct((1, _B), jnp.float32),
                   jax.ShapeDtypeStruct((1, _B), jnp.float32),
                   jax.ShapeDtypeStruct((1, 1), jnp.float32)],
    )(*args)


def _head_body(bv1_ref, s1_ref, bv2_ref, s2_ref, mts_ref, sts_ref, yc_ref,
               w1_ref, b1_ref, wmo_ref, bmo_ref, wso_ref, bso_ref,
               err2_o, meano_o):

    lane = jax.lax.broadcasted_iota(jnp.int32, (_B, _KP1), 1)

    def side_feats(bv, ysel):
        m = jnp.min(bv, axis=1, keepdims=True)
        sslot = jnp.min(jnp.where(bv == m, lane, _BIGI), axis=1, keepdims=True)
        keep = lane != sslot
        w = jnp.exp(-jnp.sqrt(jnp.maximum(bv, 0.0)))
        wk = jnp.where(keep, w, 0.0)
        sk = jnp.where(keep, ysel, 0.0)
        ws = jnp.sum(wk, axis=1, keepdims=True)
        wm = jnp.sum(sk * wk, axis=1, keepdims=True) / ws
        mean_s = jnp.sum(sk, axis=1, keepdims=True) / _K
        var = jnp.sum(jnp.where(keep, (ysel - mean_s) ** 2, 0.0),
                      axis=1, keepdims=True) / (_K - 1)
        return wm, ws, jnp.sqrt(var)

    wm1, ws1, st1 = side_feats(bv1_ref[...], s1_ref[...])
    wm2, ws2, st2 = side_feats(bv2_ref[...], s2_ref[...])
    mts = mts_ref[...]
    sts = sts_ref[...]
    feats = jnp.concatenate([wm1, ws1, st1, wm2, ws2, st2, mts, sts], axis=1)
    h = lax.dot_general(feats, w1_ref[...], (((1,), (1,)), ((), ())),
                        preferred_element_type=jnp.float32) + b1_ref[...]
    h = jnp.maximum(h, 0.0)
    mean_o = jnp.sum(h * wmo_ref[...], axis=1, keepdims=True) + bmo_ref[0, 0]
    std_o = jnp.sum(h * wso_ref[...], axis=1, keepdims=True) + bso_ref[0, 0]
    yc = yc_ref[...]
    e2 = (yc - mean_o) ** 2 / jnp.exp(std_o) + std_o
    err2_o[...] = jnp.sum(e2, axis=0, keepdims=True) / _B
    meano_o[...] = mean_o


def _head_call(bv1, s1, bv2, s2, mts, sts, ycol, w1, b1, wmo, bmo, wso, bso):
    args = [bv1, s1, bv2, s2, mts, sts, ycol, w1, b1, wmo, bmo, wso, bso]
    return pl.pallas_call(
        _head_body,
        out_shape=[jax.ShapeDtypeStruct((1, 1), jnp.float32),
                   jax.ShapeDtypeStruct((_B, 1), jnp.float32)],
    )(*args)


def kernel(params, x_left, x_right, y, index1, index2, y1_context, y2_context):
    p = params
    size1 = p['emb1'].shape[0]
    size2 = p['emb2'].shape[0]

    i1 = index1.astype(jnp.int32)
    i2 = index2.astype(jnp.int32)
    qrows1, qrows2 = _sc_row_gather(p['emb1'].reshape(-1, 128), i1 >> 2,
                                    p['emb2'].reshape(-1, 128), i2 >> 2)
    qmod1 = (i1 & 3).reshape(_B, 1)
    qmod2 = (i2 & 3).reshape(_B, 1)

    bv1, s1 = _topk(qrows1, qmod1, p['emb1'], y1_context)
    bv2, s2 = _topk(qrows2, qmod2, p['emb2'], y2_context)

    def side_w(side):
        return [p['W_ih_%s0' % side], p['W_hh_%s0' % side],
                p['b_ih_%s0' % side].reshape(3 * _H, 1),
                p['b_hh_%s0' % side].reshape(3 * _H, 1),
                p['W_ih_%s1' % side], p['W_hh_%s1' % side],
                p['b_ih_%s1' % side].reshape(3 * _H, 1),
                p['b_hh_%s1' % side].reshape(3 * _H, 1)]

    wp = (side_w('left') + side_w('right') +
          [p['W_mean'].reshape(2 * _H, 1), p['b_mean'].reshape(1, 1),
           p['W_std'].reshape(2 * _H, 1), p['b_std'].reshape(1, 1)])
    ytr = y.reshape(1, _B)
    mts_tr, sts_tr, err1 = _gru_call(x_left.T, x_right.T, wp, ytr)

    err2, mean_o = _head_call(
        bv1, s1, bv2, s2,
        mts_tr.reshape(_B, 1), sts_tr.reshape(_B, 1), y.reshape(_B, 1),
        p['W_out1'], p['b_out1'].reshape(1, _H),
        p['W_mo'], p['b_mo'].reshape(1, 1),
        p['W_so'], p['b_so'].reshape(1, 1))

    return (err1.reshape(()), err2.reshape(()), mean_o)
